# baseline (speedup 1.0000x reference)
import functools

import jax
import jax.numpy as jnp
from jax import lax
from jax.experimental import pallas as pl
from jax.experimental.pallas import tpu as pltpu
from jax.experimental.pallas import tpu_sc as plsc

_N = 10000
_E = 320000
_D = 128
_G = 64
_C = 10

_NC = 2
_NS = 16
_NW = _NC * _NS
_CHN = 128
_NCH = 80
_EPAD = _NW * _NCH * _CHN
_NP = 10240
_RT = _NP // _NS


@functools.cache
def _build_edge_agg():
    mesh = plsc.VectorSubcoreMesh(core_axis_name="c", subcore_axis_name="s")
    return pl.kernel(
        _edge_agg_body,
        out_type=jax.ShapeDtypeStruct((_NC * _NP, _D), jnp.float32),
        mesh=mesh,
        scratch_types=[
            pltpu.VMEM((2, _CHN), jnp.int32),
            pltpu.VMEM((2, _CHN), jnp.int32),
            pltpu.VMEM((2, _CHN), jnp.int32),
            pltpu.VMEM((2, _CHN), jnp.int32),
            pltpu.VMEM((_CHN, _D), jnp.float32),
            pltpu.VMEM((_CHN, _D), jnp.float32),
            pltpu.VMEM_SHARED((_NP, _D), jnp.float32),
            pltpu.SemaphoreType.DMA,
            pltpu.SemaphoreType.DMA,
            pltpu.SemaphoreType.DMA,
            pltpu.SemaphoreType.DMA,
            pltpu.SemaphoreType.DMA,
            pltpu.SemaphoreType.DMA,
        ],
    )


def _edge_agg_body(x_hbm, src_hbm, dst_hbm, zero_hbm, out_hbm,
                   ib0, ib1, ib2, ib3, rows0, rows1, acc_sh,
                   si0, si1, si2, si3, sg0, sg1):
    c = lax.axis_index("c")
    s = lax.axis_index("s")
    wid = s * _NC + c

    ibuf = (ib0, ib1, ib2, ib3)
    semi = (si0, si1, si2, si3)
    rows = (rows0, rows1)
    semg = (sg0, sg1)

    def idx_start(ci, q):
        pltpu.async_copy(src_hbm.at[wid, ci], ibuf[q].at[0], semi[q])
        pltpu.async_copy(dst_hbm.at[wid, ci], ibuf[q].at[1], semi[q])

    def idx_wait(ci, q):
        pltpu.make_async_copy(src_hbm.at[wid, ci], ibuf[q].at[0], semi[q]).wait()
        pltpu.make_async_copy(dst_hbm.at[wid, ci], ibuf[q].at[1], semi[q]).wait()

    def gather_start(b, q):
        pltpu.async_copy(x_hbm.at[ibuf[q].at[0]], rows[b], semg[b])

    def gather_wait(b, q):
        pltpu.make_async_copy(x_hbm.at[ibuf[q].at[0]], rows[b], semg[b]).wait()

    def scatter(q):
        pltpu.sync_copy(rows[q % 2], acc_sh.at[ibuf[q].at[1]], add=True)

    idx_start(0, 0)
    idx_start(1, 1)
    pltpu.sync_copy(zero_hbm, acc_sh.at[pl.ds(s * _RT, _RT)])
    idx_wait(0, 0)
    gather_start(0, 0)
    idx_wait(1, 1)
    gather_start(1, 1)
    plsc.subcore_barrier()

    def chunk_step(i, q):
        b = q % 2
        q2 = (q + 2) % 4
        gather_wait(b, q)

        @pl.when(i + 2 < _NCH)
        def _prefetch_idx():
            idx_start(i + 2, q2)

        scatter(q)

        @pl.when(i + 2 < _NCH)
        def _next_gather():
            idx_wait(i + 2, q2)
            gather_start(b, q2)

    def body(j, carry):
        for u in range(4):
            chunk_step(4 * j + u, u)
        return carry

    lax.fori_loop(0, _NCH // 4, body, 0)

    plsc.subcore_barrier()
    pltpu.sync_copy(acc_sh.at[pl.ds(s * _RT, _RT)],
                    out_hbm.at[pl.ds(c * _NP + s * _RT, _RT)])


_BLK = 1280
_NB = _NP // _BLK


def _dense_body(parts_ref, gid_ref, wg_ref, bg_ref, wm_ref, bm_ref,
                out_ref, gsum_ref):
    i = pl.program_id(0)
    agg = parts_ref[0] + parts_ref[1]
    nr = jnp.maximum(
        jnp.dot(agg, wg_ref[...], preferred_element_type=jnp.float32)
        + bg_ref[...], 0.0)
    gid = gid_ref[0, 0, :]
    onehot = (lax.broadcasted_iota(jnp.int32, (_G, _BLK), 0)
              == gid[None, :]).astype(jnp.float32)
    part = jnp.dot(onehot, nr, preferred_element_type=jnp.float32)

    @pl.when(i == 0)
    def _init():
        gsum_ref[...] = part

    @pl.when(i > 0)
    def _acc():
        gsum_ref[...] += part

    @pl.when(i == _NB - 1)
    def _fin():
        out_ref[...] = (jnp.dot(gsum_ref[...], wm_ref[...],
                                preferred_element_type=jnp.float32)
                        + bm_ref[...])


def _dense(parts, gids3, W_gnn, b_gnn, W_mlp, b_mlp):
    return pl.pallas_call(
        _dense_body,
        grid=(_NB,),
        in_specs=[
            pl.BlockSpec((2, _BLK, _D), lambda i: (0, i, 0)),
            pl.BlockSpec((1, 1, _BLK), lambda i: (i, 0, 0)),
            pl.BlockSpec((_D, _D), lambda i: (0, 0)),
            pl.BlockSpec((1, _D), lambda i: (0, 0)),
            pl.BlockSpec((_D, _C), lambda i: (0, 0)),
            pl.BlockSpec((1, _C), lambda i: (0, 0)),
        ],
        out_specs=pl.BlockSpec((_G, _C), lambda i: (0, 0)),
        out_shape=jax.ShapeDtypeStruct((_G, _C), jnp.float32),
        scratch_shapes=[pltpu.VMEM((_G, _D), jnp.float32)],
        compiler_params=pltpu.CompilerParams(
            dimension_semantics=("arbitrary",)),
    )(parts, gids3, W_gnn, b_gnn, W_mlp, b_mlp)


def kernel(x, edge_index, graph_ids, W_gnn, b_gnn, W_mlp, b_mlp):
    src = edge_index[0].astype(jnp.int32)
    dst = edge_index[1].astype(jnp.int32)
    pad = _EPAD - _E
    pad_dst = _N + jax.lax.iota(jnp.int32, pad) % (_NP - _N)
    srcp = jnp.concatenate([src, jnp.zeros((pad,), jnp.int32)])
    dstp = jnp.concatenate([dst, pad_dst])
    srcp = srcp.reshape(_NW, _NCH, _CHN)
    dstp = dstp.reshape(_NW, _NCH, _CHN)
    zero = jnp.zeros((_RT, _D), jnp.float32)
    parts = _build_edge_agg()(x, srcp, dstp, zero)
    parts = parts.reshape(_NC, _NP, _D)
    gids = jnp.concatenate([graph_ids.astype(jnp.int32),
                            jnp.full((_NP - _N,), -1, jnp.int32)])
    gids3 = gids.reshape(_NB, 1, _BLK)
    return _dense(parts, gids3, W_gnn,
                  b_gnn.reshape(1, _D), W_mlp, b_mlp.reshape(1, _C))


# speedup vs baseline: 3.0836x; 3.0836x over previous
import functools

import jax
import jax.numpy as jnp
from jax import lax
from jax.experimental import pallas as pl
from jax.experimental.pallas import tpu as pltpu
from jax.experimental.pallas import tpu_sc as plsc

_N = 10000
_E = 320000
_D = 128
_G = 64
_C = 10

_NC = 2
_NS = 16
_NW = _NC * _NS
_EW = _E // _NW
_CH = 80
_NCHUNK = _EW // _CH
_NP = 10240
_RT = _NP // _NS


@functools.cache
def _build_edge_agg():
    mesh = plsc.VectorSubcoreMesh(core_axis_name="c", subcore_axis_name="s")
    return pl.kernel(
        _edge_agg_body,
        out_type=jax.ShapeDtypeStruct((_NC * _NP, _D), jnp.float32),
        mesh=mesh,
        scratch_types=[
            pltpu.VMEM((_CH,), jnp.int32),
            pltpu.VMEM((_CH,), jnp.int32),
            pltpu.VMEM((_CH,), jnp.int32),
            pltpu.VMEM((_CH,), jnp.int32),
            pltpu.VMEM((_CH,), jnp.int32),
            pltpu.VMEM((_CH,), jnp.int32),
            pltpu.VMEM((_CH,), jnp.int32),
            pltpu.VMEM((_CH,), jnp.int32),
            pltpu.VMEM((_CH, _D), jnp.float32),
            pltpu.VMEM((_CH, _D), jnp.float32),
            pltpu.VMEM_SHARED((_NP, _D), jnp.float32),
            pltpu.SemaphoreType.DMA,
            pltpu.SemaphoreType.DMA,
            pltpu.SemaphoreType.DMA,
            pltpu.SemaphoreType.DMA,
            pltpu.SemaphoreType.DMA,
            pltpu.SemaphoreType.DMA,
        ],
    )


def _edge_agg_body(x_hbm, src_hbm, dst_hbm, zero_hbm, out_hbm,
                   sq0, sq1, sq2, sq3, dq0, dq1, dq2, dq3,
                   rows0, rows1, acc_sh,
                   si0, si1, si2, si3, sg0, sg1):
    c = lax.axis_index("c")
    s = lax.axis_index("s")
    wid = s * _NC + c
    ebase = wid * _EW

    squf = (sq0, sq1, sq2, sq3)
    dquf = (dq0, dq1, dq2, dq3)
    semi = (si0, si1, si2, si3)
    rows = (rows0, rows1)
    semg = (sg0, sg1)

    def idx_start(ci, q):
        pltpu.async_copy(src_hbm.at[pl.ds(ebase + ci * _CH, _CH)],
                         squf[q], semi[q])
        pltpu.async_copy(dst_hbm.at[pl.ds(ebase + ci * _CH, _CH)],
                         dquf[q], semi[q])

    def idx_wait(ci, q):
        pltpu.make_async_copy(src_hbm.at[pl.ds(ebase + ci * _CH, _CH)],
                              squf[q], semi[q]).wait()
        pltpu.make_async_copy(dst_hbm.at[pl.ds(ebase + ci * _CH, _CH)],
                              dquf[q], semi[q]).wait()

    def gather_start(b, q):
        pltpu.async_copy(x_hbm.at[squf[q]], rows[b], semg[b])

    def gather_wait(b, q):
        pltpu.make_async_copy(x_hbm.at[squf[q]], rows[b], semg[b]).wait()

    def scatter(q):
        pltpu.sync_copy(rows[q % 2], acc_sh.at[dquf[q]], add=True)

    idx_start(0, 0)
    idx_start(1, 1)
    pltpu.sync_copy(zero_hbm, acc_sh.at[pl.ds(s * _RT, _RT)])
    idx_wait(0, 0)
    gather_start(0, 0)
    idx_wait(1, 1)
    gather_start(1, 1)
    plsc.subcore_barrier()

    def chunk_step(i, q):
        b = q % 2
        q2 = (q + 2) % 4
        gather_wait(b, q)

        @pl.when(i + 2 < _NCHUNK)
        def _prefetch_idx():
            idx_start(i + 2, q2)

        scatter(q)

        @pl.when(i + 2 < _NCHUNK)
        def _next_gather():
            idx_wait(i + 2, q2)
            gather_start(b, q2)

    def body(j, carry):
        for u in range(4):
            chunk_step(4 * j + u, u)
        return carry

    lax.fori_loop(0, (_NCHUNK - 1) // 4, body, 0)
    gather_wait(0, 0)
    scatter(0)

    plsc.subcore_barrier()
    pltpu.sync_copy(acc_sh.at[pl.ds(s * _RT, _RT)],
                    out_hbm.at[pl.ds(c * _NP + s * _RT, _RT)])


_BLK = 1280
_NB = _NP // _BLK


def _dense_body(parts_ref, gid_ref, wg_ref, bg_ref, wm_ref, bm_ref,
                out_ref, gsum_ref):
    i = pl.program_id(0)
    agg = parts_ref[0] + parts_ref[1]
    nr = jnp.maximum(
        jnp.dot(agg, wg_ref[...], preferred_element_type=jnp.float32)
        + bg_ref[...], 0.0)
    gid = gid_ref[0, 0, :]
    onehot = (lax.broadcasted_iota(jnp.int32, (_G, _BLK), 0)
              == gid[None, :]).astype(jnp.float32)
    part = jnp.dot(onehot, nr, preferred_element_type=jnp.float32)

    @pl.when(i == 0)
    def _init():
        gsum_ref[...] = part

    @pl.when(i > 0)
    def _acc():
        gsum_ref[...] += part

    @pl.when(i == _NB - 1)
    def _fin():
        out_ref[...] = (jnp.dot(gsum_ref[...], wm_ref[...],
                                preferred_element_type=jnp.float32)
                        + bm_ref[...])


def _dense(parts, gids3, W_gnn, b_gnn, W_mlp, b_mlp):
    return pl.pallas_call(
        _dense_body,
        grid=(_NB,),
        in_specs=[
            pl.BlockSpec((2, _BLK, _D), lambda i: (0, i, 0)),
            pl.BlockSpec((1, 1, _BLK), lambda i: (i, 0, 0)),
            pl.BlockSpec((_D, _D), lambda i: (0, 0)),
            pl.BlockSpec((1, _D), lambda i: (0, 0)),
            pl.BlockSpec((_D, _C), lambda i: (0, 0)),
            pl.BlockSpec((1, _C), lambda i: (0, 0)),
        ],
        out_specs=pl.BlockSpec((_G, _C), lambda i: (0, 0)),
        out_shape=jax.ShapeDtypeStruct((_G, _C), jnp.float32),
        scratch_shapes=[pltpu.VMEM((_G, _D), jnp.float32)],
        compiler_params=pltpu.CompilerParams(
            dimension_semantics=("arbitrary",)),
    )(parts, gids3, W_gnn, b_gnn, W_mlp, b_mlp)


def kernel(x, edge_index, graph_ids, W_gnn, b_gnn, W_mlp, b_mlp):
    src = edge_index[0].astype(jnp.int32)
    dst = edge_index[1].astype(jnp.int32)
    zero = jnp.zeros((_RT, _D), jnp.float32)
    parts = _build_edge_agg()(x, src, dst, zero)
    parts = parts.reshape(_NC, _NP, _D)
    gids = jnp.concatenate([graph_ids.astype(jnp.int32),
                            jnp.full((_NP - _N,), -1, jnp.int32)])
    gids3 = gids.reshape(_NB, 1, _BLK)
    return _dense(parts, gids3, W_gnn,
                  b_gnn.reshape(1, _D), W_mlp, b_mlp.reshape(1, _C))


# speedup vs baseline: 3.6213x; 1.1744x over previous
import functools

import jax
import jax.numpy as jnp
from jax import lax
from jax.experimental import pallas as pl
from jax.experimental.pallas import tpu as pltpu
from jax.experimental.pallas import tpu_sc as plsc

_N = 10000
_E = 320000
_D = 128
_G = 64
_C = 10

_NC = 2
_NS = 16
_NW = _NC * _NS
_EW = _E // _NW
_CH = 80
_NCHUNK = _EW // _CH
_NP = 10240
_RT = _NP // _NS


@functools.cache
def _build_edge_agg():
    mesh = plsc.VectorSubcoreMesh(core_axis_name="c", subcore_axis_name="s")
    return pl.kernel(
        _edge_agg_body,
        out_type=jax.ShapeDtypeStruct((_NC * _NP, _D), jnp.float32),
        mesh=mesh,
        scratch_types=[
            [pltpu.VMEM((_CH,), jnp.int32)] * 6,
            [pltpu.VMEM((_CH,), jnp.int32)] * 6,
            [pltpu.VMEM((_CH, _D), jnp.float32)] * 3,
            pltpu.VMEM_SHARED((_NP, _D), jnp.float32),
            [pltpu.SemaphoreType.DMA] * 6,
            [pltpu.SemaphoreType.DMA] * 3,
        ],
    )


def _edge_agg_body(x_hbm, src_hbm, dst_hbm, zero_hbm, out_hbm,
                   squf, dquf, rows, acc_sh, semi, semg):
    c = lax.axis_index("c")
    s = lax.axis_index("s")
    wid = s * _NC + c
    ebase = wid * _EW

    def idx_start(ci, q):
        pltpu.async_copy(src_hbm.at[pl.ds(ebase + ci * _CH, _CH)],
                         squf[q], semi[q])
        pltpu.async_copy(dst_hbm.at[pl.ds(ebase + ci * _CH, _CH)],
                         dquf[q], semi[q])

    def idx_wait(ci, q):
        pltpu.make_async_copy(src_hbm.at[pl.ds(ebase + ci * _CH, _CH)],
                              squf[q], semi[q]).wait()
        pltpu.make_async_copy(dst_hbm.at[pl.ds(ebase + ci * _CH, _CH)],
                              dquf[q], semi[q]).wait()

    def gather_start(r, q):
        pltpu.async_copy(x_hbm.at[squf[q]], rows[r], semg[r])

    def gather_wait(r, q):
        pltpu.make_async_copy(x_hbm.at[squf[q]], rows[r], semg[r]).wait()

    def scatter(r, q):
        pltpu.sync_copy(rows[r], acc_sh.at[dquf[q]], add=True)

    idx_start(0, 0)
    idx_start(1, 1)
    idx_start(2, 2)
    pltpu.sync_copy(zero_hbm, acc_sh.at[pl.ds(s * _RT, _RT)])
    for k in range(3):
        idx_wait(k, k)
        gather_start(k, k)
    plsc.subcore_barrier()

    def chunk_step(i, u):
        r = u % 3
        q = u % 6
        q3 = (u + 3) % 6

        @pl.when(i < _NCHUNK)
        def _process():
            gather_wait(r, q)

            @pl.when(i + 3 < _NCHUNK)
            def _prefetch_idx():
                idx_start(i + 3, q3)

            scatter(r, q)

            @pl.when(i + 3 < _NCHUNK)
            def _next_gather():
                idx_wait(i + 3, q3)
                gather_start(r, q3)

    def body(j, carry):
        for u in range(6):
            chunk_step(6 * j + u, u)
        return carry

    lax.fori_loop(0, (_NCHUNK + 5) // 6, body, 0)

    plsc.subcore_barrier()
    pltpu.sync_copy(acc_sh.at[pl.ds(s * _RT, _RT)],
                    out_hbm.at[pl.ds(c * _NP + s * _RT, _RT)])


_BLK = 1280
_NB = _NP // _BLK


def _dense_body(parts_ref, gid_ref, wg_ref, bg_ref, wm_ref, bm_ref,
                out_ref, gsum_ref):
    i = pl.program_id(0)
    agg = parts_ref[0] + parts_ref[1]
    nr = jnp.maximum(
        jnp.dot(agg, wg_ref[...], preferred_element_type=jnp.float32)
        + bg_ref[...], 0.0)
    gid = gid_ref[0, 0, :]
    onehot = (lax.broadcasted_iota(jnp.int32, (_G, _BLK), 0)
              == gid[None, :]).astype(jnp.float32)
    part = jnp.dot(onehot, nr, preferred_element_type=jnp.float32)

    @pl.when(i == 0)
    def _init():
        gsum_ref[...] = part

    @pl.when(i > 0)
    def _acc():
        gsum_ref[...] += part

    @pl.when(i == _NB - 1)
    def _fin():
        out_ref[...] = (jnp.dot(gsum_ref[...], wm_ref[...],
                                preferred_element_type=jnp.float32)
                        + bm_ref[...])


def _dense(parts, gids3, W_gnn, b_gnn, W_mlp, b_mlp):
    return pl.pallas_call(
        _dense_body,
        grid=(_NB,),
        in_specs=[
            pl.BlockSpec((2, _BLK, _D), lambda i: (0, i, 0)),
            pl.BlockSpec((1, 1, _BLK), lambda i: (i, 0, 0)),
            pl.BlockSpec((_D, _D), lambda i: (0, 0)),
            pl.BlockSpec((1, _D), lambda i: (0, 0)),
            pl.BlockSpec((_D, _C), lambda i: (0, 0)),
            pl.BlockSpec((1, _C), lambda i: (0, 0)),
        ],
        out_specs=pl.BlockSpec((_G, _C), lambda i: (0, 0)),
        out_shape=jax.ShapeDtypeStruct((_G, _C), jnp.float32),
        scratch_shapes=[pltpu.VMEM((_G, _D), jnp.float32)],
        compiler_params=pltpu.CompilerParams(
            dimension_semantics=("arbitrary",)),
    )(parts, gids3, W_gnn, b_gnn, W_mlp, b_mlp)


def kernel(x, edge_index, graph_ids, W_gnn, b_gnn, W_mlp, b_mlp):
    src = edge_index[0].astype(jnp.int32)
    dst = edge_index[1].astype(jnp.int32)
    zero = jnp.zeros((_RT, _D), jnp.float32)
    parts = _build_edge_agg()(x, src, dst, zero)
    parts = parts.reshape(_NC, _NP, _D)
    gids = jnp.concatenate([graph_ids.astype(jnp.int32),
                            jnp.full((_NP - _N,), -1, jnp.int32)])
    gids3 = gids.reshape(_NB, 1, _BLK)
    return _dense(parts, gids3, W_gnn,
                  b_gnn.reshape(1, _D), W_mlp, b_mlp.reshape(1, _C))
